# baseline (speedup 1.0000x reference)
import functools

import jax
import jax.numpy as jnp
from jax import lax
from jax.experimental import pallas as pl
from jax.experimental.pallas import tpu as pltpu
from jax.experimental.pallas import tpu_sc as plsc

N = 10000
E = 320000
D = 128
DE = 16

NC = 2
NS = 16
NW = NC * NS

EP = 327680
NH = 2
EPH = EP // NH
IDXROWS = EPH // 128
RPW = IDXROWS // NW
CROWS = 2
CHUNK = CROWS * 128
NCHUNK = RPW // CROWS
CROWS_S = 1
CHUNK_S = CROWS_S * 128
NCHUNK_S = RPW // CROWS_S

JUNK = 240
NACC = N + JUNK
RPS = NACC // NS

NB = 1000
EB = 2560

_f32 = jnp.float32


def _pre_body(x_ref, w1a_ref, w1b_ref, b1_ref, pd_ref, ps_ref):
    x = x_ref[...]
    pd_ref[...] = jnp.dot(x, w1a_ref[...], preferred_element_type=_f32) + b1_ref[...]
    ps_ref[...] = jnp.dot(x, w1b_ref[...], preferred_element_type=_f32)


def _edge_mlp_body(g_ref, ea_ref, w1e_ref, w2_ref, b2_ref, m_ref):
    h = g_ref[...] + jnp.dot(ea_ref[...], w1e_ref[...], preferred_element_type=_f32)
    h = jnp.maximum(h, 0.0)
    m = jnp.dot(h, w2_ref[...], preferred_element_type=_f32) + b2_ref[...]
    m_ref[...] = jnp.maximum(m, 0.0)


def _upd_pre_body(x_ref, p_ref, q_ref, uwx_ref, uwa_ref, ub_ref,
                  w1a_ref, w1b_ref, b1_ref, x2_ref, pd_ref, ps_ref):
    a = (p_ref[0] + p_ref[1]) + (q_ref[0] + q_ref[1])
    x2 = jnp.dot(x_ref[...], uwx_ref[...], preferred_element_type=_f32)
    x2 = x2 + jnp.dot(a, uwa_ref[...], preferred_element_type=_f32) + ub_ref[...]
    x2 = jnp.maximum(x2, 0.0)
    x2_ref[...] = x2
    pd_ref[...] = jnp.dot(x2, w1a_ref[...], preferred_element_type=_f32) + b1_ref[...]
    ps_ref[...] = jnp.dot(x2, w1b_ref[...], preferred_element_type=_f32)


def _upd_ro_body(x_ref, p_ref, q_ref, uwx_ref, uwa_ref, ub_ref,
                 rw1_ref, rb1_ref, rw2_ref, rb2_ref, out_ref):
    a = (p_ref[0] + p_ref[1]) + (q_ref[0] + q_ref[1])
    x2 = jnp.dot(x_ref[...], uwx_ref[...], preferred_element_type=_f32)
    x2 = x2 + jnp.dot(a, uwa_ref[...], preferred_element_type=_f32) + ub_ref[...]
    x2 = jnp.maximum(x2, 0.0)
    h = jnp.dot(x2, rw1_ref[...], preferred_element_type=_f32) + rb1_ref[...]
    h = jnp.maximum(h, 0.0)
    out_ref[...] = jnp.dot(h, rw2_ref[...], preferred_element_type=_f32) + rb2_ref[...]


def _full(shape):
    nd = len(shape)
    return pl.BlockSpec(shape, lambda i, _nd=nd: (0,) * _nd)


def _tc_pre(x, w1a, w1b, b1):
    return pl.pallas_call(
        _pre_body,
        grid=(N // NB,),
        in_specs=[pl.BlockSpec((NB, D), lambda i: (i, 0)),
                  _full((D, D)), _full((D, D)), _full((1, D))],
        out_specs=[pl.BlockSpec((NB, D), lambda i: (i, 0))] * 2,
        out_shape=[jax.ShapeDtypeStruct((N, D), _f32)] * 2,
    )(x, w1a, w1b, b1)


def _tc_edge_mlp(g, ea, w1e, w2, b2):
    return pl.pallas_call(
        _edge_mlp_body,
        grid=(EPH // EB,),
        in_specs=[pl.BlockSpec((EB, D), lambda i: (i, 0)),
                  pl.BlockSpec((EB, DE), lambda i: (i, 0)),
                  _full((DE, D)), _full((D, D)), _full((1, D))],
        out_specs=pl.BlockSpec((EB, D), lambda i: (i, 0)),
        out_shape=jax.ShapeDtypeStruct((EPH, D), _f32),
    )(g, ea, w1e, w2, b2)


def _tc_upd_pre(x, p, q, uwx, uwa, ub, w1a, w1b, b1):
    return pl.pallas_call(
        _upd_pre_body,
        grid=(N // NB,),
        in_specs=[pl.BlockSpec((NB, D), lambda i: (i, 0)),
                  pl.BlockSpec((NC, NB, D), lambda i: (0, i, 0)),
                  pl.BlockSpec((NC, NB, D), lambda i: (0, i, 0)),
                  _full((D, D)), _full((D, D)), _full((1, D)),
                  _full((D, D)), _full((D, D)), _full((1, D))],
        out_specs=[pl.BlockSpec((NB, D), lambda i: (i, 0))] * 3,
        out_shape=[jax.ShapeDtypeStruct((N, D), _f32)] * 3,
    )(x, p, q, uwx, uwa, ub, w1a, w1b, b1)


def _tc_upd_ro(x, p, q, uwx, uwa, ub, rw1, rb1, rw2, rb2):
    return pl.pallas_call(
        _upd_ro_body,
        grid=(N // NB,),
        in_specs=[pl.BlockSpec((NB, D), lambda i: (i, 0)),
                  pl.BlockSpec((NC, NB, D), lambda i: (0, i, 0)),
                  pl.BlockSpec((NC, NB, D), lambda i: (0, i, 0)),
                  _full((D, D)), _full((D, D)), _full((1, D)),
                  _full((D, D)), _full((1, D)), _full((D, D)), _full((1, D))],
        out_specs=pl.BlockSpec((NB, D), lambda i: (i, 0)),
        out_shape=jax.ShapeDtypeStruct((N, D), _f32),
    )(x, p, q, uwx, uwa, ub, rw1, rb1, rw2, rb2)


@functools.lru_cache(maxsize=None)
def _mesh():
    return plsc.VectorSubcoreMesh(core_axis_name="c", subcore_axis_name="s")


def _sc_gather_body(pd_hbm, ps_hbm, dsti_hbm, srci_hbm, g_hbm,
                    dix_v, six_v, rows_v,
                    semi0, semi1, semp0, semp1, sema0, sema1, semo0, semo1):
    c = lax.axis_index("c")
    s = lax.axis_index("s")
    wid = s * NC + c
    rbase = wid * RPW
    semi = (semi0, semi1)
    semp = (semp0, semp1)
    sema = (sema0, sema1)
    semo = (semo0, semo1)

    def fire_idx(b, r0):
        d1 = pltpu.async_copy(dsti_hbm.at[pl.ds(r0, CROWS)], dix_v.at[b], semi[b])
        d2 = pltpu.async_copy(srci_hbm.at[pl.ds(r0, CROWS)], six_v.at[b], semi[b])
        return (d1, d2)

    def fire_plain(b):
        return [pltpu.async_copy(pd_hbm.at[dix_v.at[b, j]],
                                 rows_v.at[b, pl.ds(j * 128, 128)], semp[b])
                for j in range(CROWS)]

    def fire_add(b):
        return [pltpu.async_copy(ps_hbm.at[six_v.at[b, j]],
                                 rows_v.at[b, pl.ds(j * 128, 128)], sema[b], add=True)
                for j in range(CROWS)]

    def fire_out(b, r0):
        pltpu.async_copy(rows_v.at[b], g_hbm.at[pl.ds(r0 * 128, CHUNK)], semo[b])

    def wait_out(b, r0):
        pltpu.make_async_copy(rows_v.at[b], g_hbm.at[pl.ds(r0 * 128, CHUNK)],
                              semo[b]).wait()

    def pair(k, first):
        r0 = rbase + 2 * k * CROWS
        r1 = r0 + CROWS
        i0 = fire_idx(0, r0)
        i1 = fire_idx(1, r1)
        if not first:
            wait_out(0, r0)
        for d in i0:
            d.wait()
        p0 = fire_plain(0)
        for d in p0:
            d.wait()
        a0 = fire_add(0)
        if not first:
            wait_out(1, r1)
        for d in i1:
            d.wait()
        p1 = fire_plain(1)
        for d in a0:
            d.wait()
        fire_out(0, r0)
        for d in p1:
            d.wait()
        a1 = fire_add(1)
        for d in a1:
            d.wait()
        fire_out(1, r1)

    pair(0, True)

    def loop_body(k, carry):
        pair(k, False)
        return carry

    lax.fori_loop(1, NCHUNK // 2, loop_body, 0)
    wait_out(0, rbase)
    wait_out(1, rbase + CROWS)


@functools.lru_cache(maxsize=None)
def _sc_gather_kernel():
    return pl.kernel(
        _sc_gather_body,
        out_type=jax.ShapeDtypeStruct((EPH, D), _f32),
        mesh=_mesh(),
        scratch_types=[
            pltpu.VMEM((2, CROWS, 128), jnp.int32),
            pltpu.VMEM((2, CROWS, 128), jnp.int32),
            pltpu.VMEM((2, CHUNK, D), _f32),
            pltpu.SemaphoreType.DMA,
            pltpu.SemaphoreType.DMA,
            pltpu.SemaphoreType.DMA,
            pltpu.SemaphoreType.DMA,
            pltpu.SemaphoreType.DMA,
            pltpu.SemaphoreType.DMA,
            pltpu.SemaphoreType.DMA,
            pltpu.SemaphoreType.DMA,
        ],
    )


def _sc_gather_add(pd, ps, dst_g, src_g):
    return _sc_gather_kernel()(pd, ps, dst_g, src_g)


def _sc_scatter_body(m_hbm, dsts_hbm, z_hbm, out_hbm, idx_v, rows_v, acc_sh,
                     seml0, seml1, sems0, sems1):
    c = lax.axis_index("c")
    s = lax.axis_index("s")
    wid = s * NC + c
    rbase = wid * RPW
    seml = (seml0, seml1)

    pltpu.sync_copy(z_hbm, acc_sh.at[pl.ds(s * RPS, RPS)])
    plsc.subcore_barrier()

    def fire_loads(b, r0):
        pltpu.async_copy(dsts_hbm.at[pl.ds(r0, CROWS_S)], idx_v.at[b], seml[b])
        pltpu.async_copy(m_hbm.at[pl.ds(r0 * 128, CHUNK_S)], rows_v.at[b], seml[b])

    def wait_loads(b, r0):
        pltpu.make_async_copy(dsts_hbm.at[pl.ds(r0, CROWS_S)], idx_v.at[b],
                              seml[b]).wait()
        pltpu.make_async_copy(m_hbm.at[pl.ds(r0 * 128, CHUNK_S)], rows_v.at[b],
                              seml[b]).wait()

    def pair(k, last):
        r0 = rbase + 2 * k * CROWS_S
        r1 = r0 + CROWS_S
        wait_loads(0, r0)
        s0 = pltpu.async_copy(rows_v.at[0], acc_sh.at[idx_v.at[0, 0]], sems0,
                              add=True)
        wait_loads(1, r1)
        s1 = pltpu.async_copy(rows_v.at[1], acc_sh.at[idx_v.at[1, 0]], sems1,
                              add=True)
        s0.wait()
        if not last:
            fire_loads(0, r0 + 2 * CROWS_S)
        s1.wait()
        if not last:
            fire_loads(1, r1 + 2 * CROWS_S)

    fire_loads(0, rbase)
    fire_loads(1, rbase + CROWS_S)

    def loop_body(k, carry):
        pair(k, False)
        return carry

    lax.fori_loop(0, NCHUNK_S // 2 - 1, loop_body, 0)
    pair(NCHUNK_S // 2 - 1, True)
    plsc.subcore_barrier()
    pltpu.sync_copy(acc_sh.at[pl.ds(s * RPS, RPS)],
                    out_hbm.at[c, pl.ds(s * RPS, RPS)])


@functools.lru_cache(maxsize=None)
def _sc_segment_kernel():
    return pl.kernel(
        _sc_scatter_body,
        out_type=jax.ShapeDtypeStruct((NC, NACC, D), _f32),
        mesh=_mesh(),
        scratch_types=[
            pltpu.VMEM((2, CROWS_S, 128), jnp.int32),
            pltpu.VMEM((2, CHUNK_S, D), _f32),
            pltpu.VMEM_SHARED((NACC, D), _f32),
            pltpu.SemaphoreType.DMA,
            pltpu.SemaphoreType.DMA,
            pltpu.SemaphoreType.DMA,
            pltpu.SemaphoreType.DMA,
        ],
    )


def _sc_segment_sum(m, dst_s, zrows):
    return _sc_segment_kernel()(m, dst_s, zrows)


def kernel(x, edge_index, edge_attr, msg_w1, msg_b1, msg_w2, msg_b2,
           upd_w, upd_b, ro_w1, ro_b1, ro_w2, ro_b2):
    ei = edge_index.astype(jnp.int32)
    src = ei[0]
    dst = ei[1]

    pad = EP - E
    pad_valid = lax.iota(jnp.int32, pad) % 128
    pad_junk = N + (lax.iota(jnp.int32, pad) % JUNK)
    dst_g = jnp.concatenate([dst, pad_valid]).reshape(NH, IDXROWS, 128)
    src_g = jnp.concatenate([src, pad_valid]).reshape(NH, IDXROWS, 128)
    dst_s = jnp.concatenate([dst, pad_junk]).reshape(NH, IDXROWS, 128)
    ea_p = jnp.pad(edge_attr, ((0, pad), (0, 0))).reshape(NH, EPH, DE)
    zrows = jnp.zeros((RPS, D), _f32)

    w1a = msg_w1[:, :D, :]
    w1b = msg_w1[:, D:2 * D, :]
    w1e = msg_w1[:, 2 * D:, :]
    uwx = upd_w[:, :D, :]
    uwa = upd_w[:, D:, :]
    b1 = msg_b1.reshape(-1, 1, D)
    b2 = msg_b2.reshape(-1, 1, D)
    ub = upd_b.reshape(-1, 1, D)

    pd, ps = _tc_pre(x, w1a[0], w1b[0], b1[0])
    for l in range(3):
        g_a = _sc_gather_add(pd, ps, dst_g[0], src_g[0])
        g_b = _sc_gather_add(pd, ps, dst_g[1], src_g[1])
        m_a = _tc_edge_mlp(g_a, ea_p[0], w1e[l], msg_w2[l], b2[l])
        m_b = _tc_edge_mlp(g_b, ea_p[1], w1e[l], msg_w2[l], b2[l])
        p_a = _sc_segment_sum(m_a, dst_s[0], zrows)
        p_b = _sc_segment_sum(m_b, dst_s[1], zrows)
        if l < 2:
            x, pd, ps = _tc_upd_pre(x, p_a, p_b, uwx[l], uwa[l], ub[l],
                                    w1a[l + 1], w1b[l + 1], b1[l + 1])
        else:
            out = _tc_upd_ro(x, p_a, p_b, uwx[l], uwa[l], ub[l],
                             ro_w1, ro_b1.reshape(1, D),
                             ro_w2, ro_b2.reshape(1, D))
    return out


# speedup vs baseline: 1.0324x; 1.0324x over previous
import functools

import jax
import jax.numpy as jnp
from jax import lax
from jax.experimental import pallas as pl
from jax.experimental.pallas import tpu as pltpu
from jax.experimental.pallas import tpu_sc as plsc

N = 10000
E = 320000
D = 128
DE = 16

NC = 2
NS = 16
NW = NC * NS

EP = 327680
NH = 2
EPH = EP // NH
IDXROWS = EPH // 128
RPW = IDXROWS // NW
CROWS = 2
CHUNK = CROWS * 128
NCHUNK = RPW // CROWS
CROWS_S = 1
CHUNK_S = CROWS_S * 128
NCHUNK_S = RPW // CROWS_S

JUNK = 240
NACC = N + JUNK
RPS = NACC // NS

NB = 1000
EB = 2560

_f32 = jnp.float32


def _pre_body(x_ref, w1a_ref, w1b_ref, b1_ref, pd_ref, ps_ref):
    x = x_ref[...]
    pd_ref[...] = jnp.dot(x, w1a_ref[...], preferred_element_type=_f32) + b1_ref[...]
    ps_ref[...] = jnp.dot(x, w1b_ref[...], preferred_element_type=_f32)


def _edge_mlp_body(g_ref, ea_ref, w1e_ref, w2_ref, b2_ref, m_ref):
    h = g_ref[...] + jnp.dot(ea_ref[...], w1e_ref[...], preferred_element_type=_f32)
    h = jnp.maximum(h, 0.0)
    m = jnp.dot(h, w2_ref[...], preferred_element_type=_f32) + b2_ref[...]
    m_ref[...] = jnp.maximum(m, 0.0)


def _upd_pre_body(x_ref, p_ref, q_ref, uwx_ref, uwa_ref, ub_ref,
                  w1a_ref, w1b_ref, b1_ref, x2_ref, pd_ref, ps_ref):
    a = (p_ref[0] + p_ref[1]) + (q_ref[0] + q_ref[1])
    x2 = jnp.dot(x_ref[...], uwx_ref[...], preferred_element_type=_f32)
    x2 = x2 + jnp.dot(a, uwa_ref[...], preferred_element_type=_f32) + ub_ref[...]
    x2 = jnp.maximum(x2, 0.0)
    x2_ref[...] = x2
    pd_ref[...] = jnp.dot(x2, w1a_ref[...], preferred_element_type=_f32) + b1_ref[...]
    ps_ref[...] = jnp.dot(x2, w1b_ref[...], preferred_element_type=_f32)


def _upd_ro_body(x_ref, p_ref, q_ref, uwx_ref, uwa_ref, ub_ref,
                 rw1_ref, rb1_ref, rw2_ref, rb2_ref, out_ref):
    a = (p_ref[0] + p_ref[1]) + (q_ref[0] + q_ref[1])
    x2 = jnp.dot(x_ref[...], uwx_ref[...], preferred_element_type=_f32)
    x2 = x2 + jnp.dot(a, uwa_ref[...], preferred_element_type=_f32) + ub_ref[...]
    x2 = jnp.maximum(x2, 0.0)
    h = jnp.dot(x2, rw1_ref[...], preferred_element_type=_f32) + rb1_ref[...]
    h = jnp.maximum(h, 0.0)
    out_ref[...] = jnp.dot(h, rw2_ref[...], preferred_element_type=_f32) + rb2_ref[...]


def _full(shape):
    nd = len(shape)
    return pl.BlockSpec(shape, lambda i, _nd=nd: (0,) * _nd)


def _tc_pre(x, w1a, w1b, b1):
    return pl.pallas_call(
        _pre_body,
        grid=(N // NB,),
        in_specs=[pl.BlockSpec((NB, D), lambda i: (i, 0)),
                  _full((D, D)), _full((D, D)), _full((1, D))],
        out_specs=[pl.BlockSpec((NB, D), lambda i: (i, 0))] * 2,
        out_shape=[jax.ShapeDtypeStruct((N, D), _f32)] * 2,
    )(x, w1a, w1b, b1)


def _tc_edge_mlp(g, ea, w1e, w2, b2):
    return pl.pallas_call(
        _edge_mlp_body,
        grid=(EPH // EB,),
        in_specs=[pl.BlockSpec((EB, D), lambda i: (i, 0)),
                  pl.BlockSpec((EB, DE), lambda i: (i, 0)),
                  _full((DE, D)), _full((D, D)), _full((1, D))],
        out_specs=pl.BlockSpec((EB, D), lambda i: (i, 0)),
        out_shape=jax.ShapeDtypeStruct((EPH, D), _f32),
    )(g, ea, w1e, w2, b2)


def _tc_upd_pre(x, p, q, uwx, uwa, ub, w1a, w1b, b1):
    return pl.pallas_call(
        _upd_pre_body,
        grid=(N // NB,),
        in_specs=[pl.BlockSpec((NB, D), lambda i: (i, 0)),
                  pl.BlockSpec((NC, NB, D), lambda i: (0, i, 0)),
                  pl.BlockSpec((NC, NB, D), lambda i: (0, i, 0)),
                  _full((D, D)), _full((D, D)), _full((1, D)),
                  _full((D, D)), _full((D, D)), _full((1, D))],
        out_specs=[pl.BlockSpec((NB, D), lambda i: (i, 0))] * 3,
        out_shape=[jax.ShapeDtypeStruct((N, D), _f32)] * 3,
    )(x, p, q, uwx, uwa, ub, w1a, w1b, b1)


def _tc_upd_ro(x, p, q, uwx, uwa, ub, rw1, rb1, rw2, rb2):
    return pl.pallas_call(
        _upd_ro_body,
        grid=(N // NB,),
        in_specs=[pl.BlockSpec((NB, D), lambda i: (i, 0)),
                  pl.BlockSpec((NC, NB, D), lambda i: (0, i, 0)),
                  pl.BlockSpec((NC, NB, D), lambda i: (0, i, 0)),
                  _full((D, D)), _full((D, D)), _full((1, D)),
                  _full((D, D)), _full((1, D)), _full((D, D)), _full((1, D))],
        out_specs=pl.BlockSpec((NB, D), lambda i: (i, 0)),
        out_shape=jax.ShapeDtypeStruct((N, D), _f32),
    )(x, p, q, uwx, uwa, ub, rw1, rb1, rw2, rb2)


@functools.lru_cache(maxsize=None)
def _mesh():
    return plsc.VectorSubcoreMesh(core_axis_name="c", subcore_axis_name="s")


def _sc_gather_body(pd_hbm, ps_hbm, dsti_hbm, srci_hbm, g_hbm,
                    dix_v, six_v, rows_v,
                    semi0, semi1, semp0, semp1, sema0, sema1, semo0, semo1):
    c = lax.axis_index("c")
    s = lax.axis_index("s")
    wid = s * NC + c
    rbase = wid * RPW
    semi = (semi0, semi1)
    semp = (semp0, semp1)
    sema = (sema0, sema1)
    semo = (semo0, semo1)

    def fire_idx(b, r0):
        d1 = pltpu.async_copy(dsti_hbm.at[pl.ds(r0, CROWS)], dix_v.at[b], semi[b])
        d2 = pltpu.async_copy(srci_hbm.at[pl.ds(r0, CROWS)], six_v.at[b], semi[b])
        return (d1, d2)

    def fire_plain(b):
        return [pltpu.async_copy(pd_hbm.at[dix_v.at[b, j]],
                                 rows_v.at[b, pl.ds(j * 128, 128)], semp[b])
                for j in range(CROWS)]

    def fire_add(b):
        return [pltpu.async_copy(ps_hbm.at[six_v.at[b, j]],
                                 rows_v.at[b, pl.ds(j * 128, 128)], sema[b], add=True)
                for j in range(CROWS)]

    def fire_out(b, r0):
        pltpu.async_copy(rows_v.at[b], g_hbm.at[pl.ds(r0 * 128, CHUNK)], semo[b])

    def wait_out(b, r0):
        pltpu.make_async_copy(rows_v.at[b], g_hbm.at[pl.ds(r0 * 128, CHUNK)],
                              semo[b]).wait()

    def pair(k, first):
        r0 = rbase + 2 * k * CROWS
        r1 = r0 + CROWS
        i0 = fire_idx(0, r0)
        i1 = fire_idx(1, r1)
        if not first:
            wait_out(0, r0)
        for d in i0:
            d.wait()
        p0 = fire_plain(0)
        for d in p0:
            d.wait()
        a0 = fire_add(0)
        if not first:
            wait_out(1, r1)
        for d in i1:
            d.wait()
        p1 = fire_plain(1)
        for d in a0:
            d.wait()
        fire_out(0, r0)
        for d in p1:
            d.wait()
        a1 = fire_add(1)
        for d in a1:
            d.wait()
        fire_out(1, r1)

    pair(0, True)

    def loop_body(k, carry):
        pair(k, False)
        return carry

    lax.fori_loop(1, NCHUNK // 2, loop_body, 0)
    wait_out(0, rbase)
    wait_out(1, rbase + CROWS)


@functools.lru_cache(maxsize=None)
def _sc_gather_kernel():
    return pl.kernel(
        _sc_gather_body,
        out_type=jax.ShapeDtypeStruct((EPH, D), _f32),
        mesh=_mesh(),
        scratch_types=[
            pltpu.VMEM((2, CROWS, 128), jnp.int32),
            pltpu.VMEM((2, CROWS, 128), jnp.int32),
            pltpu.VMEM((2, CHUNK, D), _f32),
            pltpu.SemaphoreType.DMA,
            pltpu.SemaphoreType.DMA,
            pltpu.SemaphoreType.DMA,
            pltpu.SemaphoreType.DMA,
            pltpu.SemaphoreType.DMA,
            pltpu.SemaphoreType.DMA,
            pltpu.SemaphoreType.DMA,
            pltpu.SemaphoreType.DMA,
        ],
    )


def _sc_gather_add(pd, ps, dst_g, src_g):
    return _sc_gather_kernel()(pd, ps, dst_g, src_g)


def _sc_scatter_body(m_hbm, dsts_hbm, z_hbm, out_hbm, idx_v, rows_v, acc_sh,
                     seml0, seml1):
    c = lax.axis_index("c")
    s = lax.axis_index("s")
    wid = s * NC + c
    rbase = wid * RPW
    seml = (seml0, seml1)

    pltpu.sync_copy(z_hbm, acc_sh.at[pl.ds(s * RPS, RPS)])
    plsc.subcore_barrier()

    def fire_loads(b, r0):
        pltpu.async_copy(dsts_hbm.at[pl.ds(r0, CROWS_S)], idx_v.at[b], seml[b])
        pltpu.async_copy(m_hbm.at[pl.ds(r0 * 128, CHUNK_S)], rows_v.at[b], seml[b])

    def wait_loads(b, r0):
        pltpu.make_async_copy(dsts_hbm.at[pl.ds(r0, CROWS_S)], idx_v.at[b],
                              seml[b]).wait()
        pltpu.make_async_copy(m_hbm.at[pl.ds(r0 * 128, CHUNK_S)], rows_v.at[b],
                              seml[b]).wait()

    def pair(k, last):
        r0 = rbase + 2 * k * CROWS_S
        r1 = r0 + CROWS_S
        wait_loads(0, r0)
        pltpu.sync_copy(rows_v.at[0], acc_sh.at[idx_v.at[0, 0]], add=True)
        if not last:
            fire_loads(0, r0 + 2 * CROWS_S)
        wait_loads(1, r1)
        pltpu.sync_copy(rows_v.at[1], acc_sh.at[idx_v.at[1, 0]], add=True)
        if not last:
            fire_loads(1, r1 + 2 * CROWS_S)

    fire_loads(0, rbase)
    fire_loads(1, rbase + CROWS_S)

    def loop_body(k, carry):
        pair(k, False)
        return carry

    lax.fori_loop(0, NCHUNK_S // 2 - 1, loop_body, 0)
    pair(NCHUNK_S // 2 - 1, True)
    plsc.subcore_barrier()
    pltpu.sync_copy(acc_sh.at[pl.ds(s * RPS, RPS)],
                    out_hbm.at[c, pl.ds(s * RPS, RPS)])


@functools.lru_cache(maxsize=None)
def _sc_segment_kernel():
    return pl.kernel(
        _sc_scatter_body,
        out_type=jax.ShapeDtypeStruct((NC, NACC, D), _f32),
        mesh=_mesh(),
        scratch_types=[
            pltpu.VMEM((2, CROWS_S, 128), jnp.int32),
            pltpu.VMEM((2, CHUNK_S, D), _f32),
            pltpu.VMEM_SHARED((NACC, D), _f32),
            pltpu.SemaphoreType.DMA,
            pltpu.SemaphoreType.DMA,
        ],
    )


def _sc_segment_sum(m, dst_s, zrows):
    return _sc_segment_kernel()(m, dst_s, zrows)


def kernel(x, edge_index, edge_attr, msg_w1, msg_b1, msg_w2, msg_b2,
           upd_w, upd_b, ro_w1, ro_b1, ro_w2, ro_b2):
    ei = edge_index.astype(jnp.int32)
    src = ei[0]
    dst = ei[1]

    pad = EP - E
    pad_valid = lax.iota(jnp.int32, pad) % 128
    pad_junk = N + (lax.iota(jnp.int32, pad) % JUNK)
    dst_g = jnp.concatenate([dst, pad_valid]).reshape(NH, IDXROWS, 128)
    src_g = jnp.concatenate([src, pad_valid]).reshape(NH, IDXROWS, 128)
    dst_s = jnp.concatenate([dst, pad_junk]).reshape(NH, IDXROWS, 128)
    ea_p = jnp.pad(edge_attr, ((0, pad), (0, 0))).reshape(NH, EPH, DE)
    zrows = jnp.zeros((RPS, D), _f32)

    w1a = msg_w1[:, :D, :]
    w1b = msg_w1[:, D:2 * D, :]
    w1e = msg_w1[:, 2 * D:, :]
    uwx = upd_w[:, :D, :]
    uwa = upd_w[:, D:, :]
    b1 = msg_b1.reshape(-1, 1, D)
    b2 = msg_b2.reshape(-1, 1, D)
    ub = upd_b.reshape(-1, 1, D)

    pd, ps = _tc_pre(x, w1a[0], w1b[0], b1[0])
    for l in range(3):
        g_a = _sc_gather_add(pd, ps, dst_g[0], src_g[0])
        g_b = _sc_gather_add(pd, ps, dst_g[1], src_g[1])
        m_a = _tc_edge_mlp(g_a, ea_p[0], w1e[l], msg_w2[l], b2[l])
        m_b = _tc_edge_mlp(g_b, ea_p[1], w1e[l], msg_w2[l], b2[l])
        p_a = _sc_segment_sum(m_a, dst_s[0], zrows)
        p_b = _sc_segment_sum(m_b, dst_s[1], zrows)
        if l < 2:
            x, pd, ps = _tc_upd_pre(x, p_a, p_b, uwx[l], uwa[l], ub[l],
                                    w1a[l + 1], w1b[l + 1], b1[l + 1])
        else:
            out = _tc_upd_ro(x, p_a, p_b, uwx[l], uwa[l], ub[l],
                             ro_w1, ro_b1.reshape(1, D),
                             ro_w2, ro_b2.reshape(1, D))
    return out
